# baseline (speedup 1.0000x reference)
import functools

import jax
import jax.numpy as jnp
from jax import lax
from jax.experimental import pallas as pl
from jax.experimental.pallas import tpu as pltpu
from jax.experimental.pallas import tpu_sc as plsc

_N_FIELDS = 26
_VOCAB = 100000
_DIM = 16
_BATCH = 16384
_LANES = 16

_NC = 2
_NS = 16
_NW = _NC * _NS
_BW = _BATCH // _NW
_BLK = 8
_NBLK = _N_FIELDS * _VOCAB // _BLK
_CH = 32
_NCH = _BW // _CH
_NT = _N_FIELDS * _NCH


def _body(tab_hbm, xt_hbm, out_hbm, xv, rv, buf0, buf1, acc, sem):
    wid = lax.axis_index("s") * _NC + lax.axis_index("c")
    base = wid * _BW
    pltpu.sync_copy(xt_hbm.at[:, pl.ds(base, _BW)], xv)

    def prep(i, c):
        f = i // (_BW // _LANES)
        col = (i % (_BW // _LANES)) * _LANES
        v = xv[f, pl.ds(col, _LANES)]
        rv[f, pl.ds(col, _LANES)] = (v & 7) * _DIM
        xv[f, pl.ds(col, _LANES)] = (v >> 3) + f * (_VOCAB // _BLK)
        return c

    lax.fori_loop(0, _N_FIELDS * (_BW // _LANES), prep, 0)

    zeros = jnp.zeros((_LANES,), jnp.float32)

    def zero(i, c):
        acc[i // (_BW // _LANES), pl.ds((i % (_BW // _LANES)) * _LANES, _LANES)] = zeros
        return c

    lax.fori_loop(0, _DIM * (_BW // _LANES), zero, 0)

    bufs = (buf0, buf1)
    iota = lax.broadcasted_iota(jnp.int32, (_LANES,), 0)

    def fire(t, dst):
        f = t // _NCH
        c = t % _NCH
        return pltpu.async_copy(
            tab_hbm.at[xv.at[f, pl.ds(c * _CH, _CH)]], dst, sem)

    def wait(dst):
        pltpu.make_async_copy(tab_hbm.at[pl.ds(0, _CH)], dst, sem).wait()

    def process(t, src):
        f = t // _NCH
        c = t % _NCH
        for g in range(_CH // _LANES):
            lane0 = g * _LANES
            row16 = iota + lane0
            col16 = rv[f, pl.ds(c * _CH + lane0, _LANES)]
            for d in range(_DIM):
                vals = plsc.load_gather(src, [row16, col16 + d])
                plsc.addupdate(acc.at[d, pl.ds(c * _CH + lane0, _LANES)], vals)

    fire(0, buf0)

    def step(t2, c):
        for b in range(2):
            t = t2 * 2 + b
            fire(jnp.minimum(t + 1, _NT - 1), bufs[1 - b])
            wait(bufs[b])
            process(t, bufs[b])
        return c

    lax.fori_loop(0, _NT // 2, step, 0)
    wait(buf0)

    pltpu.sync_copy(acc, out_hbm.at[:, pl.ds(base, _BW)])


@functools.lru_cache(maxsize=None)
def _build_emb():
    return functools.partial(
        pl.kernel,
        out_type=jax.ShapeDtypeStruct((_DIM, _BATCH), jnp.float32),
        mesh=plsc.VectorSubcoreMesh(core_axis_name="c", subcore_axis_name="s"),
        compiler_params=pltpu.CompilerParams(
            use_tc_tiling_on_sc=False, needs_layout_passes=False),
        scratch_types=[
            pltpu.VMEM((_N_FIELDS, _BW), jnp.int32),
            pltpu.VMEM((_N_FIELDS, _BW), jnp.int32),
            pltpu.VMEM((_CH, _BLK * _DIM), jnp.float32),
            pltpu.VMEM((_CH, _BLK * _DIM), jnp.float32),
            pltpu.VMEM((_DIM, _BW), jnp.float32),
            pltpu.SemaphoreType.DMA,
        ],
    )(_body)


@jax.jit
def kernel(x, tables):
    tab_rows = tables.reshape(_NBLK, _BLK * _DIM)
    out_t = _build_emb()(tab_rows, x.T)
    return out_t.T


# speedup vs baseline: 1.1527x; 1.1527x over previous
import functools

import jax
import jax.numpy as jnp
from jax import lax
from jax.experimental import pallas as pl
from jax.experimental.pallas import tpu as pltpu
from jax.experimental.pallas import tpu_sc as plsc

_N_FIELDS = 26
_VOCAB = 100000
_DIM = 16
_BATCH = 16384
_LANES = 16

_NC = 2
_NS = 16
_NW = _NC * _NS
_BW = _BATCH // _NW
_BLK = 8
_ROWS = _VOCAB // _BLK
_NBLK = _N_FIELDS * _ROWS
_CH = 32
_NCH = _BW // _CH
_NT = _N_FIELDS * _NCH


def _body(tab_hbm, xt_hbm, out_hbm, xv, rv, buf0, buf1, acc, sem):
    wid = lax.axis_index("s") * _NC + lax.axis_index("c")
    base = wid * _BW
    pltpu.sync_copy(xt_hbm.at[:, pl.ds(base, _BW)], xv)

    def prep(i, c):
        f = i // (_BW // _LANES)
        col = (i % (_BW // _LANES)) * _LANES
        v = xv[f, pl.ds(col, _LANES)]
        rv[f, pl.ds(col, _LANES)] = (v // _ROWS) * _DIM
        xv[f, pl.ds(col, _LANES)] = (v % _ROWS) + f * _ROWS
        return c

    lax.fori_loop(0, _N_FIELDS * (_BW // _LANES), prep, 0)

    zeros = jnp.zeros((_LANES,), jnp.float32)

    def zero(i, c):
        acc[i // (_BW // _LANES), pl.ds((i % (_BW // _LANES)) * _LANES, _LANES)] = zeros
        return c

    lax.fori_loop(0, _DIM * (_BW // _LANES), zero, 0)

    bufs = (buf0, buf1)
    iota = lax.broadcasted_iota(jnp.int32, (_LANES,), 0)

    def fire(t, dst):
        f = t // _NCH
        c = t % _NCH
        return pltpu.async_copy(
            tab_hbm.at[xv.at[f, pl.ds(c * _CH, _CH)]], dst, sem)

    def wait(dst):
        pltpu.make_async_copy(tab_hbm.at[pl.ds(0, _CH)], dst, sem).wait()

    def process(t, src):
        f = t // _NCH
        c = t % _NCH
        for g in range(_CH // _LANES):
            lane0 = g * _LANES
            row16 = iota + lane0
            col16 = rv[f, pl.ds(c * _CH + lane0, _LANES)]
            for d in range(_DIM):
                vals = plsc.load_gather(src, [row16, col16 + d])
                plsc.addupdate(acc.at[d, pl.ds(c * _CH + lane0, _LANES)], vals)

    fire(0, buf0)

    def step(t2, c):
        for b in range(2):
            t = t2 * 2 + b
            fire(jnp.minimum(t + 1, _NT - 1), bufs[1 - b])
            wait(bufs[b])
            process(t, bufs[b])
        return c

    lax.fori_loop(0, _NT // 2, step, 0)
    wait(buf0)

    pltpu.sync_copy(acc, out_hbm.at[:, pl.ds(base, _BW)])


def _relayout_body(a_ref, out_ref):
    x = a_ref[0]
    for g in range(_BLK):
        piece = lax.slice(x, (0, g * _ROWS), (_DIM, (g + 1) * _ROWS))
        out_ref[0, :, pl.ds(g * _DIM, _DIM)] = piece.T


@functools.lru_cache(maxsize=None)
def _build_relayout():
    return pl.pallas_call(
        _relayout_body,
        grid=(_N_FIELDS,),
        in_specs=[pl.BlockSpec((1, _DIM, _VOCAB), lambda f: (f, 0, 0))],
        out_specs=pl.BlockSpec((1, _VOCAB // _BLK, _BLK * _DIM),
                               lambda f: (f, 0, 0)),
        out_shape=jax.ShapeDtypeStruct(
            (_N_FIELDS, _VOCAB // _BLK, _BLK * _DIM), jnp.float32),
    )


@functools.lru_cache(maxsize=None)
def _build_emb():
    return functools.partial(
        pl.kernel,
        out_type=jax.ShapeDtypeStruct((_DIM, _BATCH), jnp.float32),
        mesh=plsc.VectorSubcoreMesh(core_axis_name="c", subcore_axis_name="s"),
        compiler_params=pltpu.CompilerParams(
            use_tc_tiling_on_sc=False, needs_layout_passes=False),
        scratch_types=[
            pltpu.VMEM((_N_FIELDS, _BW), jnp.int32),
            pltpu.VMEM((_N_FIELDS, _BW), jnp.int32),
            pltpu.VMEM((_CH, _BLK * _DIM), jnp.float32),
            pltpu.VMEM((_CH, _BLK * _DIM), jnp.float32),
            pltpu.VMEM((_DIM, _BW), jnp.float32),
            pltpu.SemaphoreType.DMA,
        ],
    )(_body)


@jax.jit
def kernel(x, tables):
    tab_rows = _build_relayout()(tables.transpose(0, 2, 1)).reshape(
        _NBLK, _BLK * _DIM)
    out_t = _build_emb()(tab_rows, x.T)
    return out_t.T


# speedup vs baseline: 1.3717x; 1.1900x over previous
import functools

import jax
import jax.numpy as jnp
from jax import lax
from jax.experimental import pallas as pl
from jax.experimental.pallas import tpu as pltpu
from jax.experimental.pallas import tpu_sc as plsc

_N_FIELDS = 26
_VOCAB = 100000
_DIM = 16
_BATCH = 16384
_LANES = 16

_NC = 2
_NS = 16
_NW = _NC * _NS
_BW = _BATCH // _NW
_CH = 128
_NCHUNK = _BW // _CH

_GRP = 8
_ROWS = _VOCAB // _GRP
_NBLK = _N_FIELDS * _ROWS
_FPB = 2
_TCG = _N_FIELDS // _FPB


def _relayout_body(a_ref, out_ref):
    eye = jnp.eye(_DIM, dtype=jnp.float32)
    for fi in range(_FPB):
        for g in range(_GRP):
            piece = lax.slice(a_ref[fi], (0, g * _ROWS),
                              (_DIM, (g + 1) * _ROWS))
            piece_t = lax.dot_general(
                piece, eye, (((0,), (0,)), ((), ())),
                preferred_element_type=jnp.float32)
            out_ref[pl.ds(fi * _ROWS, _ROWS), pl.ds(g * _DIM, _DIM)] = piece_t


@functools.lru_cache(maxsize=None)
def _build_relayout():
    return pl.pallas_call(
        _relayout_body,
        grid=(_TCG,),
        in_specs=[pl.BlockSpec((_FPB, _DIM, _VOCAB), lambda i: (i, 0, 0))],
        out_specs=pl.BlockSpec((_FPB * _ROWS, _GRP * _DIM), lambda i: (i, 0)),
        out_shape=jax.ShapeDtypeStruct((_NBLK, _GRP * _DIM), jnp.float32),
    )


def _emb_body(tab_hbm, xt_hbm, out_hbm, xv, rows_v, out_v, sem):
    wid = lax.axis_index("s") * _NC + lax.axis_index("c")
    base = wid * _BW
    pltpu.sync_copy(xt_hbm.at[:, pl.ds(base, _BW)], xv)

    def chunk(g, carry):
        copies = []
        for f in range(_N_FIELDS):
            copies.append(pltpu.async_copy(
                tab_hbm.at[f].at[xv.at[f, pl.ds(g * _CH, _CH)]],
                rows_v.at[f],
                sem,
            ))
        for cp in copies:
            cp.wait()

        def accum(i, c):
            acc = rows_v[0, i, :]
            for f in range(1, _N_FIELDS):
                acc = acc + rows_v[f, i, :]
            out_v[g * _CH + i, :] = acc
            return c

        lax.fori_loop(0, _CH, accum, 0)
        return carry

    lax.fori_loop(0, _NCHUNK, chunk, 0)
    pltpu.sync_copy(out_v, out_hbm.at[pl.ds(base, _BW)])


@functools.lru_cache(maxsize=None)
def _build_emb():
    return functools.partial(
        pl.kernel,
        out_type=jax.ShapeDtypeStruct((_BATCH, _DIM), jnp.float32),
        mesh=plsc.VectorSubcoreMesh(core_axis_name="c", subcore_axis_name="s"),
        compiler_params=pltpu.CompilerParams(use_tc_tiling_on_sc=False),
        scratch_types=[
            pltpu.VMEM((_N_FIELDS, _BW), jnp.int32),
            pltpu.VMEM((_N_FIELDS, _CH, _DIM), jnp.float32),
            pltpu.VMEM((_BW, _DIM), jnp.float32),
            pltpu.SemaphoreType.DMA,
        ],
    )(_emb_body)


@jax.jit
def kernel(x, tables):
    tab_rows = _build_relayout()(tables.transpose(0, 2, 1))
    tab_flat = tab_rows.reshape(_N_FIELDS, _VOCAB, _DIM)
    xm = (x % _ROWS) * _GRP + x // _ROWS
    return _build_emb()(tab_flat, xm.T)


# speedup vs baseline: 2.1874x; 1.5946x over previous
import functools

import jax
import jax.numpy as jnp
from jax import lax
from jax.experimental import pallas as pl
from jax.experimental.pallas import tpu as pltpu
from jax.experimental.pallas import tpu_sc as plsc

_N_FIELDS = 26
_VOCAB = 100000
_DIM = 16
_BATCH = 16384
_LANES = 16

_NC = 2
_NS = 16
_NW = _NC * _NS
_BW = _BATCH // _NW
_CH = 128
_NCHUNK = _BW // _CH

_GRP = 8
_ROWS = _VOCAB // _GRP
_NBLK = _N_FIELDS * _ROWS
_FPB = 2
_TCG = _N_FIELDS // _FPB


def _relayout_body(a_hbm, out_ref, a_ref, sem):
    i = pl.program_id(0)
    pltpu.async_copy(a_hbm.at[pl.ds(i * _FPB, _FPB)], a_ref, sem).wait()
    eye = jnp.eye(_DIM, dtype=jnp.float32)
    for fi in range(_FPB):
        acc = None
        for g in range(_GRP):
            piece = a_ref[fi, :, pl.ds(g * _ROWS, _ROWS)]
            sel = jnp.pad(eye, ((0, 0), (g * _DIM, 128 - (g + 1) * _DIM)))
            part = lax.dot_general(
                piece, sel, (((0,), (0,)), ((), ())),
                preferred_element_type=jnp.float32)
            acc = part if acc is None else acc + part
        out_ref[pl.ds(fi * _ROWS, _ROWS), :] = acc


@functools.lru_cache(maxsize=None)
def _build_relayout():
    return pl.pallas_call(
        _relayout_body,
        grid=(_TCG,),
        in_specs=[pl.BlockSpec(memory_space=pl.ANY)],
        out_specs=pl.BlockSpec((_FPB * _ROWS, _GRP * _DIM), lambda i: (i, 0)),
        out_shape=jax.ShapeDtypeStruct((_NBLK, _GRP * _DIM), jnp.float32),
        scratch_shapes=[
            pltpu.VMEM((_FPB, _DIM, _VOCAB), jnp.float32),
            pltpu.SemaphoreType.DMA,
        ],
    )


def _emb_body(tab_hbm, xt_hbm, out_hbm, xv, rows_v, out_v, sem):
    wid = lax.axis_index("s") * _NC + lax.axis_index("c")
    base = wid * _BW
    pltpu.sync_copy(xt_hbm.at[:, pl.ds(base, _BW)], xv)

    def chunk(g, carry):
        copies = []
        for f in range(_N_FIELDS):
            copies.append(pltpu.async_copy(
                tab_hbm.at[f].at[xv.at[f, pl.ds(g * _CH, _CH)]],
                rows_v.at[f],
                sem,
            ))
        for cp in copies:
            cp.wait()

        def accum(i, c):
            acc = rows_v[0, i, :]
            for f in range(1, _N_FIELDS):
                acc = acc + rows_v[f, i, :]
            out_v[g * _CH + i, :] = acc
            return c

        lax.fori_loop(0, _CH, accum, 0)
        return carry

    lax.fori_loop(0, _NCHUNK, chunk, 0)
    pltpu.sync_copy(out_v, out_hbm.at[pl.ds(base, _BW)])


@functools.lru_cache(maxsize=None)
def _build_emb():
    return functools.partial(
        pl.kernel,
        out_type=jax.ShapeDtypeStruct((_BATCH, _DIM), jnp.float32),
        mesh=plsc.VectorSubcoreMesh(core_axis_name="c", subcore_axis_name="s"),
        compiler_params=pltpu.CompilerParams(use_tc_tiling_on_sc=False),
        scratch_types=[
            pltpu.VMEM((_N_FIELDS, _BW), jnp.int32),
            pltpu.VMEM((_N_FIELDS, _CH, _DIM), jnp.float32),
            pltpu.VMEM((_BW, _DIM), jnp.float32),
            pltpu.SemaphoreType.DMA,
        ],
    )(_emb_body)


@jax.jit
def kernel(x, tables):
    tab_rows = _build_relayout()(tables.transpose(0, 2, 1))
    tab_flat = tab_rows.reshape(_N_FIELDS, _VOCAB, _DIM)
    xm = (x % _ROWS) * _GRP + x // _ROWS
    return _build_emb()(tab_flat, xm.T)


# speedup vs baseline: 3.2341x; 1.4786x over previous
import functools

import jax
import jax.numpy as jnp
from jax import lax
from jax.experimental import pallas as pl
from jax.experimental.pallas import tpu as pltpu
from jax.experimental.pallas import tpu_sc as plsc

_N_FIELDS = 26
_VOCAB = 100000
_DIM = 16
_BATCH = 16384
_LANES = 16

_NC = 2
_NS = 16
_NW = _NC * _NS
_BW = _BATCH // _NW
_CH = 128
_NCHUNK = _BW // _CH

_GRP = 8
_ROWS = _VOCAB // _GRP
_NBLK = _N_FIELDS * _ROWS
_FPB = 2
_TCG = _N_FIELDS // _FPB


def _relayout_body(a_hbm, out_ref, a_ref, sem):
    i = pl.program_id(0)
    pltpu.async_copy(a_hbm.at[pl.ds(i * _FPB, _FPB)], a_ref, sem).wait()
    eye = jnp.eye(_DIM, dtype=jnp.bfloat16)
    for fi in range(_FPB):
        acc = None
        for g in range(_GRP):
            piece = a_ref[fi, :, pl.ds(g * _ROWS, _ROWS)]
            sel = jnp.pad(eye, ((0, 0), (g * _DIM, 128 - (g + 1) * _DIM)))
            part = lax.dot_general(
                piece.astype(jnp.bfloat16), sel, (((0,), (0,)), ((), ())),
                preferred_element_type=jnp.float32)
            acc = part if acc is None else acc + part
        out_ref[pl.ds(fi * _ROWS, _ROWS), :] = acc


@functools.lru_cache(maxsize=None)
def _build_relayout():
    return pl.pallas_call(
        _relayout_body,
        grid=(_TCG,),
        in_specs=[pl.BlockSpec(memory_space=pl.ANY)],
        out_specs=pl.BlockSpec((_FPB * _ROWS, _GRP * _DIM), lambda i: (i, 0)),
        out_shape=jax.ShapeDtypeStruct((_NBLK, _GRP * _DIM), jnp.float32),
        scratch_shapes=[
            pltpu.VMEM((_FPB, _DIM, _VOCAB), jnp.float32),
            pltpu.SemaphoreType.DMA,
        ],
    )


def _emb_body(tab_hbm, xt_hbm, out_hbm, xv, rows_v, out_v, sem):
    wid = lax.axis_index("s") * _NC + lax.axis_index("c")
    base = wid * _BW
    pltpu.sync_copy(xt_hbm.at[:, pl.ds(base, _BW)], xv)

    def chunk(g, carry):
        copies = []
        for f in range(_N_FIELDS):
            copies.append(pltpu.async_copy(
                tab_hbm.at[f].at[xv.at[f, pl.ds(g * _CH, _CH)]],
                rows_v.at[f],
                sem,
            ))
        for cp in copies:
            cp.wait()

        def accum(i, c):
            acc = rows_v[0, i, :]
            for f in range(1, _N_FIELDS):
                acc = acc + rows_v[f, i, :]
            out_v[g * _CH + i, :] = acc
            return c

        lax.fori_loop(0, _CH, accum, 0)
        return carry

    lax.fori_loop(0, _NCHUNK, chunk, 0)
    pltpu.sync_copy(out_v, out_hbm.at[pl.ds(base, _BW)])


@functools.lru_cache(maxsize=None)
def _build_emb():
    return functools.partial(
        pl.kernel,
        out_type=jax.ShapeDtypeStruct((_BATCH, _DIM), jnp.float32),
        mesh=plsc.VectorSubcoreMesh(core_axis_name="c", subcore_axis_name="s"),
        compiler_params=pltpu.CompilerParams(use_tc_tiling_on_sc=False),
        scratch_types=[
            pltpu.VMEM((_N_FIELDS, _BW), jnp.int32),
            pltpu.VMEM((_N_FIELDS, _CH, _DIM), jnp.float32),
            pltpu.VMEM((_BW, _DIM), jnp.float32),
            pltpu.SemaphoreType.DMA,
        ],
    )(_emb_body)


@jax.jit
def kernel(x, tables):
    tab_rows = _build_relayout()(tables.transpose(0, 2, 1))
    tab_flat = tab_rows.reshape(_N_FIELDS, _VOCAB, _DIM)
    xm = (x % _ROWS) * _GRP + x // _ROWS
    return _build_emb()(tab_flat, xm.T)
